# baseline (speedup 1.0000x reference)
import functools

import jax
import jax.numpy as jnp
from jax import lax
from jax.experimental import pallas as pl
from jax.experimental.pallas import tpu as pltpu
from jax.experimental.pallas import tpu_sc as plsc

N_USERS = 50000
N_ITEMS = 20000
N_ENTITIES = 50000
N_REL = 17
EMB = 64
HOPS = 2
B = 4096
NEG = 16
MARGIN = 0.8
DECAY = 1e-4
ANGLE_W = 0.5
ANGLE_DROP = 0.5

NCORES = 2
NTILES = 16
EP = 819200
SUPER = 1024
CHUNK = 128
NSUPER = EP // (NTILES * SUPER)

HALF_E = 25088
HALF_I = 10112
ACC_E = HALF_E + 64
ACC_I = HALF_I + 64
GARB = 63

_f32 = jnp.float32
_i32 = jnp.int32


@functools.lru_cache(maxsize=None)
def _seg_op(n_half, acc_rows, mode, with_deg):
    mesh = plsc.VectorSubcoreMesh(core_axis_name="c", subcore_axis_name="s")
    out_type = [jax.ShapeDtypeStruct((NCORES, n_half, EMB), _f32)]
    if with_deg:
        out_type.append(jax.ShapeDtypeStruct((NCORES, n_half, 8), _f32))
    scratch = [
        pltpu.VMEM((2, SUPER), _i32),
        pltpu.VMEM((2, SUPER), _f32 if mode == "scalar" else _i32),
        pltpu.VMEM((2, SUPER // CHUNK, CHUNK), _i32),
        pltpu.VMEM((2, CHUNK, EMB), _f32),
        pltpu.VMEM_SHARED((acc_rows, EMB), _f32),
        pltpu.SemaphoreType.DMA,
        pltpu.SemaphoreType.DMA,
        pltpu.SemaphoreType.DMA,
        pltpu.SemaphoreType.DMA,
        pltpu.SemaphoreType.DMA,
        pltpu.SemaphoreType.DMA,
    ]
    if mode == "rel":
        scratch.append(pltpu.VMEM((N_REL - 1, EMB), _f32))
    if with_deg:
        scratch.append(pltpu.VMEM((CHUNK, 8), _f32))
        scratch.append(pltpu.VMEM_SHARED((acc_rows, 8), _f32))

    def body(*refs):
        it = iter(refs)
        table = next(it); src = next(it); dst = next(it); w = next(it)
        rel = next(it) if mode == "rel" else None
        ones_h = next(it) if with_deg else None
        z64 = next(it); z8 = next(it)
        out = next(it)
        dout = next(it) if with_deg else None
        sbuf = next(it); wbuf = next(it); dbuf = next(it); rows = next(it)
        acc = next(it)
        isems = (next(it), next(it))
        gsems = (next(it), next(it))
        ssems = (next(it), next(it))
        relbuf = next(it) if mode == "rel" else None
        onesbuf = next(it) if with_deg else None
        dacc = next(it) if with_deg else None

        c = lax.axis_index("c")
        s = lax.axis_index("s")

        zfull = acc_rows // 2048
        ztail = acc_rows - zfull * 2048
        for j in range(-(-zfull // NTILES)):
            cid = j * NTILES + s

            @pl.when(cid < zfull)
            def _():
                base = cid * 2048
                pltpu.sync_copy(z64, acc.at[pl.ds(base, 2048)])
                if with_deg:
                    pltpu.sync_copy(z8, dacc.at[pl.ds(base, 2048)])
        if ztail:
            @pl.when(s == NTILES - 1)
            def _():
                base = zfull * 2048
                pltpu.sync_copy(z64.at[pl.ds(0, ztail)],
                                acc.at[pl.ds(base, ztail)])
                if with_deg:
                    pltpu.sync_copy(z8.at[pl.ds(0, ztail)],
                                    dacc.at[pl.ds(base, ztail)])
        if mode == "rel":
            pltpu.sync_copy(rel, relbuf)
        if with_deg:
            pltpu.sync_copy(ones_h, onesbuf)
        plsc.subcore_barrier()

        half_base = c * n_half
        nck = SUPER // CHUNK

        def idx_issue(j, sb):
            sg = s * NSUPER + j
            pltpu.async_copy(src.at[pl.ds(sg * SUPER, SUPER)],
                             sbuf.at[sb], isems[sb])
            pltpu.async_copy(w.at[pl.ds(sg * SUPER, SUPER)],
                             wbuf.at[sb], isems[sb])
            pltpu.async_copy(dst.at[pl.ds(sg * nck, nck)],
                             dbuf.at[sb], isems[sb])

        idx_issue(0, 0)
        idx_issue(1, 1)

        def scale_chunk(sb, cb, k):
            for g in range(CHUNK // 16):
                w16 = wbuf[sb, pl.ds(k * CHUNK + g * 16, 16)]
                for e in range(16):
                    m = g * 16 + e
                    if mode == "scalar":
                        wm = w16[e]
                        for q in range(EMB // 16):
                            sl = rows[cb, m, pl.ds(q * 16, 16)]
                            rows[cb, m, pl.ds(q * 16, 16)] = sl * wm
                    else:
                        tm = w16[e] - 1
                        for q in range(EMB // 16):
                            sl = rows[cb, m, pl.ds(q * 16, 16)]
                            rv = relbuf[tm, pl.ds(q * 16, 16)]
                            rows[cb, m, pl.ds(q * 16, 16)] = sl * rv

        def gwait(b, sem):
            pltpu.make_async_copy(table.at[pl.ds(0, CHUNK)],
                                  rows.at[b], sem).wait()

        def gissue(sb, k, b):
            pltpu.async_copy(
                table.at[sbuf.at[sb, pl.ds(k * CHUNK, CHUNK)]],
                rows.at[b], gsems[b])

        def super_pair(sp, carry):
            for sb in (0, 1):
                j = sp * 2 + sb
                pltpu.make_async_copy(src.at[pl.ds(0, SUPER)],
                                      sbuf.at[sb], isems[sb]).wait()
                pltpu.make_async_copy(w.at[pl.ds(0, SUPER)],
                                      wbuf.at[sb], isems[sb]).wait()
                pltpu.make_async_copy(dst.at[pl.ds(0, nck)],
                                      dbuf.at[sb], isems[sb]).wait()
                for r in range(nck):
                    for g in range(CHUNK // 16):
                        dv = dbuf[sb, r, pl.ds(g * 16, 16)]
                        loc = dv - half_base
                        ok = (loc >= 0) & (loc < n_half)
                        garb = n_half + (dv & GARB)
                        dbuf[sb, r, pl.ds(g * 16, 16)] = jnp.where(ok, loc,
                                                                   garb)
                gissue(sb, 0, 0)

                def chunk_pair(cp, carry2):
                    for cb in (0, 1):
                        k = cp * 2 + cb
                        ob = 1 - cb
                        gwait(cb, gsems[cb])
                        if cb == 1:
                            gwait(ob, ssems[ob])

                            @pl.when(cp < nck // 2 - 1)
                            def _():
                                gissue(sb, k + 1, ob)
                        else:
                            @pl.when(cp > 0)
                            def _():
                                gwait(ob, ssems[ob])
                            gissue(sb, k + 1, ob)
                        scale_chunk(sb, cb, k)
                        pltpu.async_copy(rows.at[cb],
                                         acc.at[dbuf.at[sb, k]],
                                         ssems[cb], add=True)
                        if with_deg:
                            pltpu.sync_copy(onesbuf, dacc.at[dbuf.at[sb, k]],
                                            add=True)
                    return carry2

                lax.fori_loop(0, nck // 2, chunk_pair, 0, unroll=False)
                gwait(1, ssems[1])
                @pl.when(sp < NSUPER // 2 - 1)
                def _():
                    idx_issue(j + 2, sb)
            return carry

        lax.fori_loop(0, NSUPER // 2, super_pair, 0, unroll=False)
        plsc.subcore_barrier()

        nchunks = n_half // CHUNK
        for j in range(-(-nchunks // NTILES)):
            cid = j * NTILES + s

            @pl.when(cid < nchunks)
            def _():
                r0 = cid * CHUNK
                pltpu.sync_copy(acc.at[pl.ds(r0, CHUNK)],
                                out.at[c, pl.ds(r0, CHUNK)])
                if with_deg:
                    pltpu.sync_copy(dacc.at[pl.ds(r0, CHUNK)],
                                    dout.at[c, pl.ds(r0, CHUNK)])

    return pl.kernel(body, out_type=out_type, mesh=mesh,
                     scratch_types=scratch,
                     compiler_params=pltpu.CompilerParams(
                         use_tc_tiling_on_sc=False))


def _seg(table, src, dst, w, n_half, acc_rows, rel=None, with_deg=False):
    z64 = jnp.zeros((2048, EMB), _f32)
    z8 = jnp.zeros((2048, 8), _f32)
    mode = "rel" if rel is not None else "scalar"
    k = _seg_op(n_half, acc_rows, mode, with_deg)
    args = [table, src, dst.reshape(EP // CHUNK, CHUNK), w]
    if rel is not None:
        args.append(rel)
    if with_deg:
        ones = jnp.zeros((CHUNK, 8), _f32).at[:, 0].set(1.0)
        args.append(ones)
    args += [z64, z8]
    out = k(*args)
    return out if isinstance(out, (tuple, list)) else (out,)


@functools.lru_cache(maxsize=None)
def _deg_op(n_half, acc_rows):
    mesh = plsc.VectorSubcoreMesh(core_axis_name="c", subcore_axis_name="s")
    nck = SUPER // CHUNK

    def body(dst, ones_h, z8, dout, dbuf, onesbuf, dacc):
        c = lax.axis_index("c")
        s = lax.axis_index("s")
        zfull = acc_rows // 2048
        ztail = acc_rows - zfull * 2048
        for j in range(-(-zfull // NTILES)):
            cid = j * NTILES + s

            @pl.when(cid < zfull)
            def _():
                pltpu.sync_copy(z8, dacc.at[pl.ds(cid * 2048, 2048)])
        if ztail:
            @pl.when(s == NTILES - 1)
            def _():
                pltpu.sync_copy(z8.at[pl.ds(0, ztail)],
                                dacc.at[pl.ds(zfull * 2048, ztail)])
        pltpu.sync_copy(ones_h, onesbuf)
        plsc.subcore_barrier()
        half_base = c * n_half

        def super_body(j, carry):
            sg = s * NSUPER + j
            pltpu.sync_copy(dst.at[pl.ds(sg * nck, nck)], dbuf)
            for r in range(nck):
                for g in range(CHUNK // 16):
                    dv = dbuf[r, pl.ds(g * 16, 16)]
                    loc = dv - half_base
                    ok = (loc >= 0) & (loc < n_half)
                    garb = n_half + (dv & GARB)
                    dbuf[r, pl.ds(g * 16, 16)] = jnp.where(ok, loc, garb)

            def chunk(k, carry2):
                pltpu.sync_copy(onesbuf, dacc.at[dbuf.at[k]], add=True)
                return carry2

            lax.fori_loop(0, nck, chunk, 0, unroll=False)
            return carry

        lax.fori_loop(0, NSUPER, super_body, 0, unroll=False)
        plsc.subcore_barrier()
        nchunks = n_half // CHUNK
        for j in range(-(-nchunks // NTILES)):
            cid = j * NTILES + s

            @pl.when(cid < nchunks)
            def _():
                r0 = cid * CHUNK
                pltpu.sync_copy(dacc.at[pl.ds(r0, CHUNK)],
                                dout.at[c, pl.ds(r0, CHUNK)])

    return pl.kernel(
        body,
        out_type=jax.ShapeDtypeStruct((NCORES, n_half, 8), _f32),
        mesh=mesh,
        scratch_types=[pltpu.VMEM((SUPER // CHUNK, CHUNK), _i32),
                       pltpu.VMEM((CHUNK, 8), _f32),
                       pltpu.VMEM_SHARED((acc_rows, 8), _f32)],
        compiler_params=pltpu.CompilerParams(use_tc_tiling_on_sc=False))


def _deg(dst, n_half, acc_rows):
    ones = jnp.zeros((CHUNK, 8), _f32).at[:, 0].set(1.0)
    z8 = jnp.zeros((2048, 8), _f32)
    return _deg_op(n_half, acc_rows)(dst.reshape(EP // CHUNK, CHUNK),
                                     ones, z8)


@functools.lru_cache(maxsize=None)
def _gather_op(n_rows, n_idx):
    mesh = plsc.VectorSubcoreMesh(core_axis_name="c", subcore_axis_name="s")
    per_tile = n_idx // (NCORES * NTILES)
    npairs = per_tile // (2 * CHUNK)

    def body(table, idx, out, ibuf, rows, gs0, gs1, ss0, ss1):
        c = lax.axis_index("c")
        s = lax.axis_index("s")
        wid = s * NCORES + c
        base = wid * per_tile
        gsems = (gs0, gs1)
        ssems = (ss0, ss1)
        pltpu.sync_copy(idx.at[pl.ds(base, per_tile)], ibuf)

        def gwait(b, sem):
            pltpu.make_async_copy(table.at[pl.ds(0, CHUNK)],
                                  rows.at[b], sem).wait()

        def gissue(k, b):
            pltpu.async_copy(table.at[ibuf.at[pl.ds(k * CHUNK, CHUNK)]],
                             rows.at[b], gsems[b])

        gissue(0, 0)

        def pair(cp, carry):
            for cb in (0, 1):
                k = cp * 2 + cb
                ob = 1 - cb
                gwait(cb, gsems[cb])
                if cb == 1:
                    gwait(ob, ssems[ob])

                    @pl.when(cp < npairs - 1)
                    def _():
                        gissue(k + 1, ob)
                else:
                    @pl.when(cp > 0)
                    def _():
                        gwait(ob, ssems[ob])
                    gissue(k + 1, ob)
                pltpu.async_copy(rows.at[cb],
                                 out.at[pl.ds(base + k * CHUNK, CHUNK)],
                                 ssems[cb])
            return carry

        lax.fori_loop(0, npairs, pair, 0, unroll=False)
        gwait(1, ssems[1])

    return pl.kernel(
        body,
        out_type=jax.ShapeDtypeStruct((n_idx, EMB), _f32),
        mesh=mesh,
        scratch_types=[pltpu.VMEM((per_tile,), _i32),
                       pltpu.VMEM((2, CHUNK, EMB), _f32),
                       pltpu.SemaphoreType.DMA, pltpu.SemaphoreType.DMA,
                       pltpu.SemaphoreType.DMA, pltpu.SemaphoreType.DMA],
        compiler_params=pltpu.CompilerParams(use_tc_tiling_on_sc=False))


def _gather(table, idx):
    return _gather_op(table.shape[0], idx.shape[0])(table, idx)


def _t1_body(agg_ref, deg_ref, res_ref, e1_ref, out_ref):
    d = jnp.maximum(deg_ref[:, 0:1], 1.0)
    x = agg_ref[...] / d
    e1_ref[...] = x
    nrm = jnp.sqrt(jnp.sum(x * x, axis=1, keepdims=True)) + 1e-8
    out_ref[...] = res_ref[...] + x / nrm


def _t2_body(x_ref, res_ref, out_ref):
    x = x_ref[...]
    nrm = jnp.sqrt(jnp.sum(x * x, axis=1, keepdims=True)) + 1e-8
    out_ref[...] = res_ref[...] + x / nrm


@functools.lru_cache(maxsize=None)
def _t1_call(n_rows):
    blk = 512
    grid = n_rows // blk
    return pl.pallas_call(
        _t1_body,
        grid=(grid,),
        in_specs=[pl.BlockSpec((blk, EMB), lambda i: (i, 0)),
                  pl.BlockSpec((blk, 8), lambda i: (i, 0)),
                  pl.BlockSpec((blk, EMB), lambda i: (i, 0))],
        out_specs=[pl.BlockSpec((blk, EMB), lambda i: (i, 0)),
                   pl.BlockSpec((blk, EMB), lambda i: (i, 0))],
        out_shape=[jax.ShapeDtypeStruct((n_rows, EMB), _f32),
                   jax.ShapeDtypeStruct((n_rows, EMB), _f32)])


@functools.lru_cache(maxsize=None)
def _t2_call(n_rows):
    blk = 512 if n_rows % 512 == 0 else 256
    grid = n_rows // blk
    return pl.pallas_call(
        _t2_body,
        grid=(grid,),
        in_specs=[pl.BlockSpec((blk, EMB), lambda i: (i, 0)),
                  pl.BlockSpec((blk, EMB), lambda i: (i, 0))],
        out_specs=pl.BlockSpec((blk, EMB), lambda i: (i, 0)),
        out_shape=jax.ShapeDtypeStruct((n_rows, EMB), _f32))


def _loss_body(u_ref, pe_ref, pi_ref, ne_ref, ni_ref, l1_ref, sq_ref):
    i = pl.program_id(0)
    u_e = u_ref[...]
    pe = pe_ref[...]
    pi = pi_ref[...]
    ne = ne_ref[...]
    ni = ni_ref[...]
    u = u_e / (jnp.sqrt(jnp.sum(u_e * u_e, -1, keepdims=True)) + 1e-8)
    ps = pe + pi
    p = ps / (jnp.sqrt(jnp.sum(ps * ps, -1, keepdims=True)) + 1e-8)
    pos_score = jnp.sum(u * p, -1)
    ns = ne + ni
    n = ns / (jnp.sqrt(jnp.sum(ns * ns, -1, keepdims=True)) + 1e-8)
    neg_score = jnp.sum(u[:, None, :] * n, -1)
    hinge = (jnp.maximum(1.0 - pos_score, 0.0)
             + jnp.mean(jnp.maximum(neg_score - MARGIN, 0.0), -1))
    part1 = jnp.sum(hinge)
    sq = (jnp.sum(u_e * u_e) + jnp.sum(pe * pe) + jnp.sum(pi * pi)
          + jnp.sum(ne * ne) + jnp.sum(ni * ni))

    @pl.when(i == 0)
    def _():
        l1_ref[0, 0] = 0.0
        sq_ref[0, 0] = 0.0

    l1_ref[0, 0] += part1
    sq_ref[0, 0] += sq


def _loss_call():
    blk = 512
    grid = B // blk
    return pl.pallas_call(
        _loss_body,
        grid=(grid,),
        in_specs=[pl.BlockSpec((blk, EMB), lambda i: (i, 0)),
                  pl.BlockSpec((blk, EMB), lambda i: (i, 0)),
                  pl.BlockSpec((blk, EMB), lambda i: (i, 0)),
                  pl.BlockSpec((blk, NEG, EMB), lambda i: (i, 0, 0)),
                  pl.BlockSpec((blk, NEG, EMB), lambda i: (i, 0, 0))],
        out_specs=[pl.BlockSpec(memory_space=pltpu.SMEM),
                   pl.BlockSpec(memory_space=pltpu.SMEM)],
        out_shape=[jax.ShapeDtypeStruct((1, 1), _f32),
                   jax.ShapeDtypeStruct((1, 1), _f32)])


def _acos_poly(x):
    a = jnp.abs(x)
    s = jnp.sqrt(1.0 - a)
    p = jnp.float32(-0.0012624911)
    for cc in (0.0066700901, -0.0170881256, 0.0308918810, -0.0501743046,
               0.0889789874, -0.2145988016, 1.5707963050):
        p = p * a + jnp.float32(cc)
    b = s * p
    return jnp.where(x >= 0, b, jnp.float32(jnp.pi) - b)


def _angle_body(t_real, h_ref, t_ref, out_ref):
    i = pl.program_id(0)
    blk = h_ref.shape[0]
    h = h_ref[...] * ANGLE_DROP
    t = t_ref[...] * ANGLE_DROP
    eps = 1e-6
    k_const = 0.1
    hh = jnp.sum(h * h, -1)
    tt = jnp.sum(t * t, -1)
    dot = jnp.sum(h * t, -1)
    d = h - t
    edist = jnp.sqrt(jnp.sum(d * d, -1))
    nu = jnp.sqrt(hh)
    num = dot * (1.0 + hh) - hh * (1.0 + tt)
    den = nu * edist * jnp.sqrt(jnp.clip(1.0 + tt * hh - 2.0 * dot, eps)) + eps
    ang = _acos_poly(jnp.clip(num / den, -1.0 + eps, 1.0 - eps))
    sqnu = jnp.clip(hh, 0.0, 1.0 - eps)
    asin_arg = jnp.clip(k_const * (1.0 - sqnu) / jnp.sqrt(sqnu + eps),
                        -1.0 + eps, 1.0 - eps)
    half = jnp.float32(jnp.pi / 2) - _acos_poly(asin_arg)
    val = jnp.maximum(ang - half, 0.0)
    rid = i * blk + lax.broadcasted_iota(_i32, (blk,), 0)
    part = jnp.sum(jnp.where(rid < t_real, val, 0.0))

    @pl.when(i == 0)
    def _():
        out_ref[0, 0] = 0.0

    out_ref[0, 0] += part


@functools.lru_cache(maxsize=None)
def _angle_call(tp, t_real):
    blk = 2048
    grid = tp // blk
    return pl.pallas_call(
        functools.partial(_angle_body, t_real),
        grid=(grid,),
        in_specs=[pl.BlockSpec((blk, EMB), lambda i: (i, 0)),
                  pl.BlockSpec((blk, EMB), lambda i, g=grid: (g + i, 0))],
        out_specs=pl.BlockSpec(memory_space=pltpu.SMEM),
        out_shape=jax.ShapeDtypeStruct((1, 1), _f32))


def _pad_rows(x, n):
    return jnp.zeros((n, EMB), _f32).at[: x.shape[0]].set(x)


def _pad_edges(x, val, dtype):
    return jnp.concatenate(
        [x.astype(dtype), jnp.full((EP - x.shape[0],), val, dtype)])


def kernel(user, pos_item, neg_item, all_embed, item_emb_cf, rel_emb,
           edge_index, edge_type, ui_rows, ui_cols, ui_vals,
           tri_head, tri_tail):
    user_emb = all_embed[:N_USERS]
    entity_emb = all_embed[N_USERS:]

    headp = _pad_edges(edge_index[0], -1, _i32)
    tailp = _pad_edges(edge_index[1], 0, _i32)
    typep = _pad_edges(edge_type, 1, _i32)
    ur_src = _pad_edges(ui_rows, 0, _i32)
    ur_dst = _pad_edges(ui_rows, -1, _i32)
    uc_src = _pad_edges(ui_cols, 0, _i32)
    uc_dst = _pad_edges(ui_cols, -1, _i32)
    uvp = _pad_edges(ui_vals, 0.0, _f32)

    e_res = _pad_rows(entity_emb, 2 * HALF_E)
    u_res = _pad_rows(user_emb, 2 * HALF_E)
    i_res = _pad_rows(item_emb_cf, 2 * HALF_I)

    etab = entity_emb
    itab = item_emb_cf
    deg = _deg(headp, HALF_E, ACC_E).reshape(2 * HALF_E, 8)
    for hop in range(HOPS):
        (eagg,) = _seg(etab, tailp, headp, typep, HALF_E, ACC_E,
                       rel=rel_emb)
        (uagg,) = _seg(etab, uc_src, ur_dst, uvp, HALF_E, ACC_E)
        (ucf,) = _seg(itab, uc_src, ur_dst, uvp, HALF_E, ACC_E)
        (icf,) = _seg(ucf.reshape(2 * HALF_E, EMB), ur_src, uc_dst, uvp,
                      HALF_I, ACC_I)
        eagg = eagg.reshape(2 * HALF_E, EMB)
        uagg = uagg.reshape(2 * HALF_E, EMB)
        icf = icf.reshape(2 * HALF_I, EMB)
        etab, e_res = _t1_call(2 * HALF_E)(eagg, deg, e_res)
        u_res = _t2_call(2 * HALF_E)(uagg, u_res)
        i_res = _t2_call(2 * HALF_I)(icf, i_res)
        itab = icf

    neg_flat = neg_item.reshape(-1).astype(_i32)
    idx_en = jnp.concatenate([neg_flat, pos_item.astype(_i32),
                              jnp.zeros((4096,), _i32)])
    idx_u = jnp.concatenate([user.astype(_i32), jnp.zeros((4096,), _i32)])
    g_e = _gather(e_res, idx_en)
    g_i = _gather(i_res, idx_en)
    g_u = _gather(u_res, idx_u)

    ne3 = g_e[: B * NEG].reshape(B, NEG, EMB)
    ni3 = g_i[: B * NEG].reshape(B, NEG, EMB)
    pe = g_e[B * NEG: B * NEG + B]
    pi = g_i[B * NEG: B * NEG + B]
    l1, sq = _loss_call()(g_u, pe, pi, ne3, ni3)
    loss1 = l1[0, 0] / B
    reg = DECAY * sq[0, 0] / (2.0 * B)

    t_real = tri_head.shape[0]
    tp = -(-t_real // 4096) * 4096
    padh = jnp.zeros((tp - t_real,), _i32)
    idx_ht = jnp.concatenate([tri_head.astype(_i32), padh,
                              tri_tail.astype(_i32), padh])
    g_ht = _gather(entity_emb, idx_ht)
    asum = _angle_call(tp, t_real)(g_ht, g_ht)
    loss2 = ANGLE_W * asum[0, 0] / t_real

    return loss1 + reg + loss2
